# baseline (speedup 1.0000x reference)
import jax
import jax.numpy as jnp
from jax.experimental import pallas as pl
from jax.experimental.pallas import tpu as pltpu

_B, _R, _C = 4, 4096, 2048
_RBLK = 2048
_NR = _R // _RBLK
_NSPLIT = 2
_CH = _C // _NSPLIT


def _argmin_body(*refs):
    x_refs = refs[:_NSPLIT]
    o_ref, m_ref, i_ref = refs[_NSPLIT:]
    r = pl.program_id(1)
    for c, x_ref in enumerate(x_refs):
        cols = slice(c * _CH, (c + 1) * _CH)

        def scan_body(a, carry):
            amin, aidx = carry
            sl = x_ref[0, pl.ds(a * 8, 8), :]
            took = sl < amin
            return jnp.minimum(amin, sl), jnp.where(took, a, aidx)

        init = (x_ref[0, 0:8, :], jnp.zeros((8, _CH), jnp.int32))
        amin, aidx = jax.lax.fori_loop(1, _RBLK // 8, scan_body, init,
                                       unroll=4)

        rows = aidx * 8 + jax.lax.broadcasted_iota(jnp.int32, (8, _CH), 0)
        bm = jnp.min(amin, axis=0, keepdims=True)
        bidx = jnp.min(jnp.where(amin <= bm, rows, _R), axis=0,
                       keepdims=True) + r * _RBLK

        @pl.when(r == 0)
        def _init():
            m_ref[0:1, cols] = bm
            i_ref[0:1, cols] = bidx

        @pl.when(r == _NR - 1)
        def _emit():
            take = bm < m_ref[0:1, cols]
            o_ref[0, 0:1, cols] = jnp.where(take, bidx, i_ref[0:1, cols])


def kernel(x):
    in_specs = [
        pl.BlockSpec((1, _RBLK, _CH), lambda b, r, c=c: (b, r, c))
        for c in range(_NSPLIT)
    ]
    out = pl.pallas_call(
        _argmin_body,
        grid=(_B, _NR),
        in_specs=in_specs,
        out_specs=pl.BlockSpec((1, 1, _C), lambda b, r: (b, 0, 0)),
        out_shape=jax.ShapeDtypeStruct((_B, 1, _C), jnp.int32),
        scratch_shapes=[
            pltpu.VMEM((1, _C), jnp.float32),
            pltpu.VMEM((1, _C), jnp.int32),
        ],
        compiler_params=pltpu.CompilerParams(
            dimension_semantics=("parallel", "arbitrary"),
        ),
    )(*([x] * _NSPLIT))
    return out.reshape(_B, _C).astype(jnp.int64)


# speedup vs baseline: 1.0439x; 1.0439x over previous
import jax
import jax.numpy as jnp
from jax.experimental import pallas as pl
from jax.experimental.pallas import tpu as pltpu

_B, _R, _C = 4, 4096, 2048
_RBLK = 2048
_NR = _R // _RBLK
_QSPLIT = 2
_QR = _RBLK // _QSPLIT
_NCH = 2
_CH = _C // _NCH


def _argmin_body(*refs):
    x_refs = refs[:_QSPLIT]
    o_ref, m_ref, i_ref = refs[_QSPLIT:]
    r = pl.program_id(1)
    for ch in range(_NCH):
        cols = slice(ch * _CH, (ch + 1) * _CH)

        amin = x_refs[0][0, 0, 0:8, cols]
        aidx = jnp.zeros((8, _CH), jnp.int32)

        for q, x_ref in enumerate(x_refs):
            def scan_body(a, carry, x_ref=x_ref, q=q):
                amin, aidx = carry
                sl = x_ref[0, 0, pl.ds(a * 8, 8), cols]
                took = sl < amin
                return (jnp.minimum(amin, sl),
                        jnp.where(took, a + q * (_QR // 8), aidx))

            amin, aidx = jax.lax.fori_loop(1 if q == 0 else 0, _QR // 8,
                                           scan_body, (amin, aidx),
                                           unroll=4)

        rows = aidx * 8 + jax.lax.broadcasted_iota(jnp.int32, (8, _CH), 0)
        bm = jnp.min(amin, axis=0, keepdims=True)
        bidx = jnp.min(jnp.where(amin <= bm, rows, _R), axis=0,
                       keepdims=True) + r * _RBLK

        @pl.when(r == 0)
        def _init():
            m_ref[0:1, cols] = bm
            i_ref[0:1, cols] = bidx

        @pl.when(r == _NR - 1)
        def _emit():
            take = bm < m_ref[0:1, cols]
            o_ref[0, 0:1, cols] = jnp.where(take, bidx, i_ref[0:1, cols])


def kernel(x):
    xr = x.reshape(_B, _R // _QR, _QR, _C)
    in_specs = [
        pl.BlockSpec((1, 1, _QR, _C), lambda b, r, q=q: (b, _QSPLIT * r + q, 0, 0))
        for q in range(_QSPLIT)
    ]
    out = pl.pallas_call(
        _argmin_body,
        grid=(_B, _NR),
        in_specs=in_specs,
        out_specs=pl.BlockSpec((1, 1, _C), lambda b, r: (b, 0, 0)),
        out_shape=jax.ShapeDtypeStruct((_B, 1, _C), jnp.int32),
        scratch_shapes=[
            pltpu.VMEM((1, _C), jnp.float32),
            pltpu.VMEM((1, _C), jnp.int32),
        ],
        compiler_params=pltpu.CompilerParams(
            dimension_semantics=("parallel", "arbitrary"),
        ),
    )(*([xr] * _QSPLIT))
    return out.reshape(_B, _C).astype(jnp.int64)
